# Initial kernel scaffold; baseline (speedup 1.0000x reference)
import jax
import jax.numpy as jnp
from jax.experimental import pallas as pl


def kernel(x, sample_pos, edge_index, W1, b1, W2, b2, centers, cls_W, cls_b):
    raise NotImplementedError("write your pallas kernel here")


# speedup vs baseline: 730.7289x; 730.7289x over previous
import functools

import jax
import jax.numpy as jnp
from jax import lax
from jax.experimental import pallas as pl
from jax.experimental.pallas import tpu as pltpu
from jax.experimental.pallas import tpu_sc as plsc

SIGMA = 0.2
NPER = 2048
EPER = 8192
NSEG = 16
NTASK = 32
L = 16


def _sigmoid(x):
    return 1.0 / (1.0 + jnp.exp(-x))


def _mlp_body(x_ref, w1_ref, b1_ref, w2_ref, b2_ref, v_ref):
    h = _sigmoid(
        jnp.dot(x_ref[...], w1_ref[...], preferred_element_type=jnp.float32)
        + b1_ref[0, :][None, :]
    )
    u = jnp.dot(h, w2_ref[...], preferred_element_type=jnp.float32) + b2_ref[0, 0]
    v_ref[...] = _sigmoid(u[:, 0:1])


def _readout_body(px_ref, py_ref, centers_ref, clsw_ref, clsb_ref, out_ref):
    inv = 1.0 / (2.0 * SIGMA * SIGMA)
    px = px_ref[...]
    py = py_ref[...]
    logits = jnp.broadcast_to(clsb_ref[0, :][None, :], (NSEG, 10)).astype(jnp.float32)
    for c in range(64):
        cx = centers_ref[c, 0]
        cy = centers_ref[c, 1]
        dx = px - cx
        dy = py - cy
        g = jnp.exp(-(dx * dx + dy * dy) * inv)
        sc = jnp.sum(g, axis=1)
        logits = logits + sc[:NSEG, None] * clsw_ref[c, :][None, :]
        logits = logits + sc[NSEG:, None] * clsw_ref[64 + c, :][None, :]
    out_ref[...] = logits


def _ph_body(v_hbm, el_hbm, px_hbm, py_hbm,
             vv, keys0, pay0, keys1, pay1, hist, parent, tmpb, tmpd,
             pxv, pyv, eav, ebv):
    wid = lax.axis_index("s") * 2 + lax.axis_index("c")
    s = wid % NSEG
    lvl = wid // NSEG
    sgn = jnp.where(lvl == 0, 1.0, -1.0).astype(jnp.float32)
    base = (s * NPER).astype(jnp.int32)
    lanes = lax.iota(jnp.int32, L)
    onesi = jnp.ones((L,), jnp.int32)

    pltpu.sync_copy(v_hbm.at[s], vv)
    pltpu.sync_copy(el_hbm.at[s], eav)
    pltpu.sync_copy(el_hbm.at[NSEG + s], ebv)

    def key_chunk(c, _):
        av = eav[pl.ds(c * L, L)] - base
        bv = ebv[pl.ds(c * L, L)] - base
        fa = plsc.load_gather(vv, [av]) * sgn
        fb = plsc.load_gather(vv, [bv]) * sgn
        ev = jnp.maximum(fa, fb)
        u = lax.bitcast_convert_type(ev, jnp.int32)
        key = jnp.where(u < 0, ~u, u ^ jnp.int32(-2147483648))
        keys0[pl.ds(c * L, L)] = key
        pay0[pl.ds(c * L, L)] = av | (bv << 11)
        return 0

    lax.fori_loop(0, EPER // L, key_chunk, 0)

    def radix_pass(shift, src_k, src_p, dst_k, dst_p):
        def zero(c, _):
            hist[pl.ds(c * L, L)] = jnp.zeros((L,), jnp.int32)
            return 0

        lax.fori_loop(0, 4096 // L, zero, 0)

        def hist_chunk(c, _):
            idxv = lanes * (EPER // L) + c
            k = plsc.load_gather(src_k, [idxv])
            d = lax.shift_right_logical(k, shift) & 255
            plsc.addupdate_scatter(hist, [d * L + lanes], onesi)
            return 0

        lax.fori_loop(0, EPER // L, hist_chunk, 0)

        def scan_chunk(c, run):
            hv = hist[pl.ds(c * L, L)]
            cs = plsc.cumsum(hv)
            hist[pl.ds(c * L, L)] = cs - hv + run
            return run + jnp.sum(hv)

        lax.fori_loop(0, 4096 // L, scan_chunk, jnp.int32(0))

        def scat_chunk(c, _):
            idxv = lanes * (EPER // L) + c
            k = plsc.load_gather(src_k, [idxv])
            p = plsc.load_gather(src_p, [idxv])
            h = (lax.shift_right_logical(k, shift) & 255) * L + lanes
            pos = plsc.load_gather(hist, [h])
            plsc.store_scatter(hist, [h], pos + 1)
            plsc.store_scatter(dst_k, [pos], k)
            plsc.store_scatter(dst_p, [pos], p)
            return 0

        lax.fori_loop(0, EPER // L, scat_chunk, 0)

    radix_pass(0, keys0, pay0, keys1, pay1)
    radix_pass(8, keys1, pay1, keys0, pay0)
    radix_pass(16, keys0, pay0, keys1, pay1)
    radix_pass(24, keys1, pay1, keys0, pay0)

    def init_parent(c, _):
        parent[pl.ds(c * L, L)] = lanes + c * L
        return 0

    lax.fori_loop(0, NPER // L, init_parent, 0)

    lane0 = lanes == 0
    lane8 = lanes == 8
    halfmask = lanes < 8
    wr_mask = lane0 | lane8

    def edge_step(e, cnt):
        pay = plsc.load_gather(pay0, [jnp.full((L,), e, jnp.int32)])
        av = pay & 2047
        bv = lax.shift_right_logical(pay, 11) & 2047
        cur0 = jnp.where(halfmask, av, bv)
        p0 = plsc.load_gather(parent, [cur0])

        def fcond(st):
            cur, p = st
            return jnp.any(p != cur)

        def fbody(st):
            cur, p = st
            pp = plsc.load_gather(parent, [p])
            plsc.store_scatter(parent, [cur], pp, mask=wr_mask)
            return p, pp

        _, roots = lax.while_loop(fcond, fbody, (cur0, p0))
        ra = jnp.sum(jnp.where(lane0, roots, 0))
        rb = jnp.sum(jnp.where(lane8, roots, 0))
        merged = ra != rb

        q = jnp.where(lanes == 0, ra,
                      jnp.where(lanes == 1, rb,
                                jnp.where(lanes == 2, av, bv)))
        fq = plsc.load_gather(vv, [q])
        v_ra = jnp.sum(jnp.where(lanes == 0, fq, 0.0))
        v_rb = jnp.sum(jnp.where(lanes == 1, fq, 0.0))
        v_a = jnp.sum(jnp.where(lanes == 2, fq, 0.0))
        v_b = jnp.sum(jnp.where(lanes == 3, fq, 0.0))

        elder = v_ra * sgn <= v_rb * sgn
        loser = jnp.where(elder, rb, ra)
        winner = jnp.where(elder, ra, rb)
        y_val = jnp.where(elder, v_rb, v_ra)
        d_val = jnp.where(v_a * sgn >= v_b * sgn, v_a, v_b)

        mvec = lane0 & merged
        plsc.store_scatter(parent, [jnp.full((L,), loser, jnp.int32)],
                           jnp.full((L,), winner, jnp.int32), mask=mvec)
        cntv = jnp.full((L,), cnt, jnp.int32)
        plsc.store_scatter(tmpb, [cntv], jnp.full((L,), y_val, jnp.float32),
                           mask=mvec)
        plsc.store_scatter(tmpd, [cntv], jnp.full((L,), d_val, jnp.float32),
                           mask=mvec)
        return cnt + merged.astype(jnp.int32)

    cnt = lax.fori_loop(0, EPER, edge_step, jnp.int32(0))

    big = jnp.float32(1e9)

    def emit0(c, _):
        j = lanes + c * L
        valid = j < cnt
        jj = jnp.where(valid, j, 0)
        bvals = plsc.load_gather(tmpb, [jj])
        dvals = plsc.load_gather(tmpd, [jj])
        pxv[pl.ds(c * L, L)] = jnp.where(valid, bvals, big)
        pyv[pl.ds(c * L, L)] = jnp.where(valid, dvals, big)
        return 0

    def emit1(c, _):
        j = lanes + c * L
        valid = j < cnt
        k0 = 2 * j
        k1 = 2 * j + 1

        def cc_at(k):
            in_d = k < cnt
            gd = plsc.load_gather(tmpd, [jnp.where(in_d, k, 0)])
            gb = plsc.load_gather(tmpb, [jnp.where(in_d | ~valid, 0, k - cnt)])
            return jnp.where(in_d, gd, gb)

        pxv[pl.ds(c * L, L)] = jnp.where(valid, cc_at(k0), big)
        pyv[pl.ds(c * L, L)] = jnp.where(valid, cc_at(k1), big)
        return 0

    @pl.when(lvl == 0)
    def _():
        lax.fori_loop(0, NPER // L, emit0, 0)

    @pl.when(lvl != 0)
    def _():
        lax.fori_loop(0, NPER // L, emit1, 0)

    pltpu.sync_copy(pxv, px_hbm.at[wid])
    pltpu.sync_copy(pyv, py_hbm.at[wid])


def kernel(x, sample_pos, edge_index, W1, b1, W2, b2, centers, cls_W, cls_b):
    del sample_pos
    w2p = jnp.pad(W2, ((0, 0), (0, 127)))
    vcol = pl.pallas_call(
        _mlp_body,
        grid=(NSEG,),
        in_specs=[
            pl.BlockSpec((NPER, 128), lambda i: (i, 0)),
            pl.BlockSpec((128, 64), lambda i: (0, 0)),
            pl.BlockSpec((1, 64), lambda i: (0, 0)),
            pl.BlockSpec((64, 128), lambda i: (0, 0)),
            pl.BlockSpec((1, 1), lambda i: (0, 0)),
        ],
        out_specs=pl.BlockSpec((NPER, 1), lambda i: (i, 0)),
        out_shape=jax.ShapeDtypeStruct((NSEG * NPER, 1), jnp.float32),
    )(x, W1, b1.reshape(1, -1), w2p, b2.reshape(1, 1))
    v = vcol.reshape(NSEG, NPER)

    el = edge_index.astype(jnp.int32).reshape(2 * NSEG, EPER)

    mesh = plsc.VectorSubcoreMesh(core_axis_name="c", subcore_axis_name="s")
    ph = functools.partial(
        pl.kernel,
        out_type=[
            jax.ShapeDtypeStruct((NTASK, NPER), jnp.float32),
            jax.ShapeDtypeStruct((NTASK, NPER), jnp.float32),
        ],
        mesh=mesh,
        compiler_params=pltpu.CompilerParams(needs_layout_passes=False),
        scratch_types=[
            pltpu.VMEM((NPER,), jnp.float32),
            pltpu.VMEM((EPER,), jnp.int32),
            pltpu.VMEM((EPER,), jnp.int32),
            pltpu.VMEM((EPER,), jnp.int32),
            pltpu.VMEM((EPER,), jnp.int32),
            pltpu.VMEM((4096,), jnp.int32),
            pltpu.VMEM((NPER,), jnp.int32),
            pltpu.VMEM((NPER,), jnp.float32),
            pltpu.VMEM((NPER,), jnp.float32),
            pltpu.VMEM((NPER,), jnp.float32),
            pltpu.VMEM((NPER,), jnp.float32),
            pltpu.VMEM((EPER,), jnp.int32),
            pltpu.VMEM((EPER,), jnp.int32),
        ],
    )(_ph_body)
    px, py = ph(v, el)

    out = pl.pallas_call(
        _readout_body,
        out_shape=jax.ShapeDtypeStruct((NSEG, 10), jnp.float32),
    )(px, py, centers, cls_W, cls_b.reshape(1, -1))
    return out


# speedup vs baseline: 787.6393x; 1.0779x over previous
import functools

import jax
import jax.numpy as jnp
from jax import lax
from jax.experimental import pallas as pl
from jax.experimental.pallas import tpu as pltpu
from jax.experimental.pallas import tpu_sc as plsc

SIGMA = 0.2
NPER = 2048
EPER = 8192
NSEG = 16
NTASK = 32
L = 16


def _sigmoid(x):
    return 1.0 / (1.0 + jnp.exp(-x))


def _mlp_body(x_ref, w1_ref, b1_ref, w2_ref, b2_ref, v_ref):
    h = _sigmoid(
        jnp.dot(x_ref[...], w1_ref[...], preferred_element_type=jnp.float32)
        + b1_ref[0, :][None, :]
    )
    u = jnp.dot(h, w2_ref[...], preferred_element_type=jnp.float32) + b2_ref[0, 0]
    v_ref[...] = _sigmoid(u[:, 0:1])


def _readout_body(px_ref, py_ref, centers_ref, clsw_ref, clsb_ref, out_ref):
    inv = 1.0 / (2.0 * SIGMA * SIGMA)
    px = px_ref[...]
    py = py_ref[...]
    logits = jnp.broadcast_to(clsb_ref[0, :][None, :], (NSEG, 10)).astype(jnp.float32)
    for c in range(64):
        cx = centers_ref[c, 0]
        cy = centers_ref[c, 1]
        dx = px - cx
        dy = py - cy
        g = jnp.exp(-(dx * dx + dy * dy) * inv)
        sc = jnp.sum(g, axis=1)
        logits = logits + sc[:NSEG, None] * clsw_ref[c, :][None, :]
        logits = logits + sc[NSEG:, None] * clsw_ref[64 + c, :][None, :]
    out_ref[...] = logits


def _ph_body(v_hbm, el_hbm, px_hbm, py_hbm,
             vv, keys0, pay0, keys1, pay1, hist, parent, tmpp,
             pxv, pyv, eav, ebv):
    wid = lax.axis_index("s") * 2 + lax.axis_index("c")
    s = wid % NSEG
    lvl = wid // NSEG
    sgn = jnp.where(lvl == 0, 1.0, -1.0).astype(jnp.float32)
    base = (s * NPER).astype(jnp.int32)
    lanes = lax.iota(jnp.int32, L)
    onesi = jnp.ones((L,), jnp.int32)

    pltpu.sync_copy(v_hbm.at[s], vv)
    pltpu.sync_copy(el_hbm.at[s], eav)
    pltpu.sync_copy(el_hbm.at[NSEG + s], ebv)

    def key_chunk(c, _):
        av = eav[pl.ds(c * L, L)] - base
        bv = ebv[pl.ds(c * L, L)] - base
        fa = plsc.load_gather(vv, [av]) * sgn
        fb = plsc.load_gather(vv, [bv]) * sgn
        ev = jnp.maximum(fa, fb)
        u = lax.bitcast_convert_type(ev, jnp.int32)
        key = jnp.where(u < 0, ~u, u ^ jnp.int32(-2147483648))
        keys0[pl.ds(c * L, L)] = key
        pay0[pl.ds(c * L, L)] = av | (bv << 11)
        return 0

    lax.fori_loop(0, EPER // L, key_chunk, 0)

    def radix_pass(shift, src_k, src_p, dst_k, dst_p):
        def zero(c, _):
            hist[pl.ds(c * L, L)] = jnp.zeros((L,), jnp.int32)
            return 0

        lax.fori_loop(0, 4096 // L, zero, 0)

        def hist_chunk(c, _):
            idxv = lanes * (EPER // L) + c
            k = plsc.load_gather(src_k, [idxv])
            d = lax.shift_right_logical(k, shift) & 255
            plsc.addupdate_scatter(hist, [d * L + lanes], onesi)
            return 0

        lax.fori_loop(0, EPER // L, hist_chunk, 0)

        def scan_chunk(c, run):
            hv = hist[pl.ds(c * L, L)]
            cs = plsc.cumsum(hv)
            hist[pl.ds(c * L, L)] = cs - hv + run
            return run + jnp.sum(hv)

        lax.fori_loop(0, 4096 // L, scan_chunk, jnp.int32(0))

        def scat_chunk(c, _):
            idxv = lanes * (EPER // L) + c
            k = plsc.load_gather(src_k, [idxv])
            p = plsc.load_gather(src_p, [idxv])
            h = (lax.shift_right_logical(k, shift) & 255) * L + lanes
            pos = plsc.load_gather(hist, [h])
            plsc.store_scatter(hist, [h], pos + 1)
            plsc.store_scatter(dst_k, [pos], k)
            plsc.store_scatter(dst_p, [pos], p)
            return 0

        lax.fori_loop(0, EPER // L, scat_chunk, 0)

    radix_pass(0, keys0, pay0, keys1, pay1)
    radix_pass(8, keys1, pay1, keys0, pay0)
    radix_pass(16, keys0, pay0, keys1, pay1)
    radix_pass(24, keys1, pay1, keys0, pay0)

    def init_parent(c, _):
        parent[pl.ds(c * L, L)] = lanes + c * L
        return 0

    lax.fori_loop(0, NPER // L, init_parent, 0)

    lane0 = lanes == 0
    lane8 = lanes == 8
    halfmask = lanes < 8
    wr_mask = lane0 | lane8
    pair_mask = lanes < 2

    def edge_step(e, cnt):
        ev = jnp.full((L,), e, jnp.int32)
        pay = plsc.load_gather(pay0, [ev])
        key = plsc.load_gather(keys0, [ev])
        av = pay & 2047
        bv = lax.shift_right_logical(pay, 11) & 2047
        cur0 = jnp.where(halfmask, av, bv)
        p0 = plsc.load_gather(parent, [cur0])

        def fcond(st):
            cur, p = st
            return jnp.any(p != cur)

        def fbody(st):
            cur, p = st
            pp = plsc.load_gather(parent, [p])
            plsc.store_scatter(parent, [cur], pp, mask=wr_mask)
            return p, pp

        _, roots = lax.while_loop(fcond, fbody, (cur0, p0))
        ra = jnp.sum(jnp.where(lane0, roots, 0))
        rb = jnp.sum(jnp.where(lane8, roots, 0))
        merged = ra != rb

        fq = plsc.load_gather(vv, [roots])
        v_ra = jnp.sum(jnp.where(lane0, fq, 0.0))
        v_rb = jnp.sum(jnp.where(lane8, fq, 0.0))

        u = jnp.where(key < 0, key ^ jnp.int32(-2147483648), ~key)
        d_val = jnp.sum(jnp.where(lane0,
                                  lax.bitcast_convert_type(u, jnp.float32), 0.0)) * sgn

        elder = v_ra * sgn <= v_rb * sgn
        loser = jnp.where(elder, rb, ra)
        winner = jnp.where(elder, ra, rb)
        y_val = jnp.where(elder, v_rb, v_ra)

        mvec = lane0 & merged
        plsc.store_scatter(parent, [jnp.full((L,), loser, jnp.int32)],
                           jnp.full((L,), winner, jnp.int32), mask=mvec)
        pidx = 2 * cnt + lanes
        pv = jnp.where(lane0, y_val, d_val)
        plsc.store_scatter(tmpp, [pidx], pv, mask=pair_mask & merged)
        return cnt + merged.astype(jnp.int32)

    cnt = lax.fori_loop(0, EPER, edge_step, jnp.int32(0))

    big = jnp.float32(1e9)

    def emit0(c, _):
        j = lanes + c * L
        valid = j < cnt
        jj = jnp.where(valid, j, 0)
        bvals = plsc.load_gather(tmpp, [2 * jj])
        dvals = plsc.load_gather(tmpp, [2 * jj + 1])
        pxv[pl.ds(c * L, L)] = jnp.where(valid, bvals, big)
        pyv[pl.ds(c * L, L)] = jnp.where(valid, dvals, big)
        return 0

    def emit1(c, _):
        j = lanes + c * L
        valid = j < cnt
        k0 = 2 * j
        k1 = 2 * j + 1

        def cc_at(k):
            in_d = k < cnt
            gd = plsc.load_gather(tmpp, [jnp.where(in_d, 2 * k + 1, 1)])
            gb = plsc.load_gather(tmpp, [jnp.where(in_d | ~valid, 0,
                                                   2 * (k - cnt))])
            return jnp.where(in_d, gd, gb)

        pxv[pl.ds(c * L, L)] = jnp.where(valid, cc_at(k0), big)
        pyv[pl.ds(c * L, L)] = jnp.where(valid, cc_at(k1), big)
        return 0

    @pl.when(lvl == 0)
    def _():
        lax.fori_loop(0, NPER // L, emit0, 0)

    @pl.when(lvl != 0)
    def _():
        lax.fori_loop(0, NPER // L, emit1, 0)

    pltpu.sync_copy(pxv, px_hbm.at[wid])
    pltpu.sync_copy(pyv, py_hbm.at[wid])


def kernel(x, sample_pos, edge_index, W1, b1, W2, b2, centers, cls_W, cls_b):
    del sample_pos
    w2p = jnp.pad(W2, ((0, 0), (0, 127)))
    vcol = pl.pallas_call(
        _mlp_body,
        grid=(NSEG,),
        in_specs=[
            pl.BlockSpec((NPER, 128), lambda i: (i, 0)),
            pl.BlockSpec((128, 64), lambda i: (0, 0)),
            pl.BlockSpec((1, 64), lambda i: (0, 0)),
            pl.BlockSpec((64, 128), lambda i: (0, 0)),
            pl.BlockSpec((1, 1), lambda i: (0, 0)),
        ],
        out_specs=pl.BlockSpec((NPER, 1), lambda i: (i, 0)),
        out_shape=jax.ShapeDtypeStruct((NSEG * NPER, 1), jnp.float32),
    )(x, W1, b1.reshape(1, -1), w2p, b2.reshape(1, 1))
    v = vcol.reshape(NSEG, NPER)

    el = edge_index.astype(jnp.int32).reshape(2 * NSEG, EPER)

    mesh = plsc.VectorSubcoreMesh(core_axis_name="c", subcore_axis_name="s")
    ph = functools.partial(
        pl.kernel,
        out_type=[
            jax.ShapeDtypeStruct((NTASK, NPER), jnp.float32),
            jax.ShapeDtypeStruct((NTASK, NPER), jnp.float32),
        ],
        mesh=mesh,
        compiler_params=pltpu.CompilerParams(needs_layout_passes=False),
        scratch_types=[
            pltpu.VMEM((NPER,), jnp.float32),
            pltpu.VMEM((EPER,), jnp.int32),
            pltpu.VMEM((EPER,), jnp.int32),
            pltpu.VMEM((EPER,), jnp.int32),
            pltpu.VMEM((EPER,), jnp.int32),
            pltpu.VMEM((4096,), jnp.int32),
            pltpu.VMEM((NPER,), jnp.int32),
            pltpu.VMEM((2 * NPER,), jnp.float32),
            pltpu.VMEM((NPER,), jnp.float32),
            pltpu.VMEM((NPER,), jnp.float32),
            pltpu.VMEM((EPER,), jnp.int32),
            pltpu.VMEM((EPER,), jnp.int32),
        ],
    )(_ph_body)
    px, py = ph(v, el)

    out = pl.pallas_call(
        _readout_body,
        out_shape=jax.ShapeDtypeStruct((NSEG, 10), jnp.float32),
    )(px, py, centers, cls_W, cls_b.reshape(1, -1))
    return out
